# baseline (speedup 1.0000x reference)
import functools

import jax
import jax.numpy as jnp
import numpy as np
from jax.experimental import pallas as pl

B = 64
K_SEL = 32
D = 8192
TAU0 = 0.5
_TINY = float(np.finfo(np.float32).tiny)

_KS0 = 0
_KS1 = 42
_KS2 = 42 ^ 0x1BD11BDA

_ROT1 = (13, 15, 26, 6)
_ROT2 = (17, 29, 16, 24)


def _round_group(x, y, rots):
    for r in rots:
        x = x + y
        y = (y << r) | (y >> (32 - r))
        y = x ^ y
    return x, y


def _threefry_xor(y):
    u32 = lambda v: jnp.uint32(v)
    x = jnp.zeros_like(y)
    y = y + u32(_KS1)
    x, y = _round_group(x, y, _ROT1)
    x = x + u32(_KS1)
    y = y + u32(_KS2 + 1)
    x, y = _round_group(x, y, _ROT2)
    x = x + u32(_KS2)
    y = y + u32(_KS0 + 2)
    x, y = _round_group(x, y, _ROT1)
    x = x + u32(_KS0)
    y = y + u32(_KS1 + 3)
    x, y = _round_group(x, y, _ROT2)
    x = x + u32(_KS1)
    y = y + u32(_KS2 + 4)
    x, y = _round_group(x, y, _ROT1)
    x = x + u32(_KS2)
    y = y + u32(_KS0 + 5)
    return x ^ y


def _block_kernel(logits_ref, out_ref, *, block_w):
    j = pl.program_id(0)
    c0 = (j * block_w).astype(jnp.uint32)
    row = jax.lax.broadcasted_iota(jnp.uint32, (B, block_w), 0)
    col = jax.lax.broadcasted_iota(jnp.uint32, (B, block_w), 1) + c0
    base = row * jnp.uint32(K_SEL * D) + col

    sign = jnp.int32(-(2**31))

    def kstep(k, m):
        bits = _threefry_xor(base + k.astype(jnp.uint32) * jnp.uint32(D))
        return jnp.maximum(m, jax.lax.bitcast_convert_type(bits, jnp.int32) ^ sign)

    m = jax.lax.fori_loop(0, K_SEL, kstep, jnp.full((B, block_w), sign, jnp.int32))
    m = jax.lax.bitcast_convert_type(m ^ sign, jnp.uint32)

    fb = (m >> 9) | jnp.uint32(0x3F800000)
    f = jax.lax.bitcast_convert_type(fb, jnp.float32) - 1.0
    u = jnp.maximum(jnp.float32(_TINY), f * jnp.float32(1.0 - _TINY) + jnp.float32(_TINY))
    g = -jnp.log(-jnp.log(u))
    out_ref[...] = jax.nn.sigmoid((g + logits_ref[...]) * jnp.float32(1.0 / TAU0))


@jax.jit
def kernel(logits):
    block_w = 1024
    grid = (D // block_w,)
    return pl.pallas_call(
        functools.partial(_block_kernel, block_w=block_w),
        grid=grid,
        in_specs=[pl.BlockSpec((B, block_w), lambda j: (0, j))],
        out_specs=pl.BlockSpec((B, block_w), lambda j: (0, j)),
        out_shape=jax.ShapeDtypeStruct((B, D), jnp.float32),
    )(logits)


# speedup vs baseline: 1.5361x; 1.5361x over previous
import functools

import jax
import jax.numpy as jnp
import numpy as np
from jax.experimental import pallas as pl

B = 64
K_SEL = 32
D = 8192
TAU0 = 0.5
_TINY = float(np.finfo(np.float32).tiny)

_KS0 = 0
_KS1 = 42
_KS2 = 42 ^ 0x1BD11BDA

_ROT1 = (13, 15, 26, 6)
_ROT2 = (17, 29, 16, 24)


def _round_group(x, y, rots):
    for r in rots:
        x = x + y
        y = (y << r) | (y >> (32 - r))
        y = x ^ y
    return x, y


def _threefry_xor(y):
    u32 = lambda v: jnp.uint32(v)
    x = jnp.zeros_like(y)
    y = y + u32(_KS1)
    x, y = _round_group(x, y, _ROT1)
    x = x + u32(_KS1)
    y = y + u32(_KS2 + 1)
    x, y = _round_group(x, y, _ROT2)
    x = x + u32(_KS2)
    y = y + u32(_KS0 + 2)
    x, y = _round_group(x, y, _ROT1)
    x = x + u32(_KS0)
    y = y + u32(_KS1 + 3)
    x, y = _round_group(x, y, _ROT2)
    x = x + u32(_KS1)
    y = y + u32(_KS2 + 4)
    x, y = _round_group(x, y, _ROT1)
    x = x + u32(_KS2)
    y = y + u32(_KS0 + 5)
    return x ^ y


def _block_kernel(logits_ref, out_ref, *, block_w, rg):
    j = pl.program_id(0)
    c0 = (j * block_w).astype(jnp.uint32)
    row = jax.lax.broadcasted_iota(jnp.uint32, (rg, block_w), 0)
    col = jax.lax.broadcasted_iota(jnp.uint32, (rg, block_w), 1) + c0

    sign = jnp.int32(-(2**31))

    def row_group(g, _):
        b0 = (g * rg).astype(jnp.uint32)
        base = (row + b0) * jnp.uint32(K_SEL * D) + col

        def kstep(k, m):
            bits = _threefry_xor(base + k.astype(jnp.uint32) * jnp.uint32(D))
            return jnp.maximum(m, jax.lax.bitcast_convert_type(bits, jnp.int32) ^ sign)

        m = jax.lax.fori_loop(0, K_SEL, kstep,
                              jnp.full((rg, block_w), sign, jnp.int32),
                              unroll=32)
        m = jax.lax.bitcast_convert_type(m ^ sign, jnp.uint32)

        fb = (m >> 9) | jnp.uint32(0x3F800000)
        f = jax.lax.bitcast_convert_type(fb, jnp.float32) - 1.0
        u = jnp.maximum(jnp.float32(_TINY),
                        f * jnp.float32(1.0 - _TINY) + jnp.float32(_TINY))
        gum = -jnp.log(-jnp.log(u))
        lg = logits_ref[pl.ds(g * rg, rg), :]
        out_ref[pl.ds(g * rg, rg), :] = jax.nn.sigmoid(
            (gum + lg) * jnp.float32(1.0 / TAU0))
        return 0

    jax.lax.fori_loop(0, B // rg, row_group, 0)


@jax.jit
def kernel(logits):
    block_w = 1024
    rg = 16
    grid = (D // block_w,)
    return pl.pallas_call(
        functools.partial(_block_kernel, block_w=block_w, rg=rg),
        grid=grid,
        in_specs=[pl.BlockSpec((B, block_w), lambda j: (0, j))],
        out_specs=pl.BlockSpec((B, block_w), lambda j: (0, j)),
        out_shape=jax.ShapeDtypeStruct((B, D), jnp.float32),
    )(logits)


# speedup vs baseline: 1.5923x; 1.0366x over previous
import functools

import jax
import jax.numpy as jnp
import numpy as np
from jax import lax
from jax.experimental import pallas as pl
from jax.experimental.pallas import tpu as pltpu
from jax.experimental.pallas import tpu_sc as plsc

B = 64
K_SEL = 32
D = 8192
TAU0 = 0.5
_TINY = float(np.finfo(np.float32).tiny)

W_SC = 1024
D_TC = D - W_SC
NW = 32
ROWS_PER_W = B // NW
CH = 4

_KS0 = 0
_KS1 = 42
_KS2 = 42 ^ 0x1BD11BDA

_ROT1 = (13, 15, 26, 6)
_ROT2 = (17, 29, 16, 24)


def _round_group(x, y, rots):
    for r in rots:
        x = x + y
        y = (y << r) | (y >> (32 - r))
        y = x ^ y
    return x, y


def _threefry_xor(y):
    u32 = lambda v: jnp.uint32(v)
    x = jnp.zeros_like(y)
    y = y + u32(_KS1)
    x, y = _round_group(x, y, _ROT1)
    x = x + u32(_KS1)
    y = y + u32(_KS2 + 1)
    x, y = _round_group(x, y, _ROT2)
    x = x + u32(_KS2)
    y = y + u32(_KS0 + 2)
    x, y = _round_group(x, y, _ROT1)
    x = x + u32(_KS0)
    y = y + u32(_KS1 + 3)
    x, y = _round_group(x, y, _ROT2)
    x = x + u32(_KS1)
    y = y + u32(_KS2 + 4)
    x, y = _round_group(x, y, _ROT1)
    x = x + u32(_KS2)
    y = y + u32(_KS0 + 5)
    return x ^ y


def _block_kernel(logits_ref, out_ref, *, block_w, rg):
    j = pl.program_id(0)
    c0 = (j * block_w).astype(jnp.uint32)
    row = jax.lax.broadcasted_iota(jnp.uint32, (rg, block_w), 0)
    col = jax.lax.broadcasted_iota(jnp.uint32, (rg, block_w), 1) + c0

    sign = jnp.int32(-(2**31))

    def row_group(g, _):
        b0 = (g * rg).astype(jnp.uint32)
        base = (row + b0) * jnp.uint32(K_SEL * D) + col

        def kstep(k, m):
            bits = _threefry_xor(base + k.astype(jnp.uint32) * jnp.uint32(D))
            return jnp.maximum(m, jax.lax.bitcast_convert_type(bits, jnp.int32) ^ sign)

        m = jax.lax.fori_loop(0, K_SEL, kstep,
                              jnp.full((rg, block_w), sign, jnp.int32),
                              unroll=K_SEL)
        m = jax.lax.bitcast_convert_type(m ^ sign, jnp.uint32)

        fb = (m >> 9) | jnp.uint32(0x3F800000)
        f = jax.lax.bitcast_convert_type(fb, jnp.float32) - 1.0
        u = jnp.maximum(jnp.float32(_TINY),
                        f * jnp.float32(1.0 - _TINY) + jnp.float32(_TINY))
        gum = -jnp.log(-jnp.log(u))
        lg = logits_ref[pl.ds(g * rg, rg), :]
        out_ref[pl.ds(g * rg, rg), :] = jax.nn.sigmoid(
            (gum + lg) * jnp.float32(1.0 / TAU0))
        return 0

    jax.lax.fori_loop(0, B // rg, row_group, 0)


def _tc_part(logits_tc):
    block_w = 1024
    rg = 16
    grid = (D_TC // block_w,)
    return pl.pallas_call(
        functools.partial(_block_kernel, block_w=block_w, rg=rg),
        grid=grid,
        in_specs=[pl.BlockSpec((B, block_w), lambda j: (0, j))],
        out_specs=pl.BlockSpec((B, block_w), lambda j: (0, j)),
        out_shape=jax.ShapeDtypeStruct((B, D_TC), jnp.float32),
    )(logits_tc)


_LN2_HI = 0.693359375
_LN2_LO = -2.12194440e-4
_SQRT2 = 1.4142135623730951


def _poly_log(x):
    xb = jax.lax.bitcast_convert_type(x, jnp.int32)
    e = (xb >> 23) - jnp.int32(127)
    mb = (xb & jnp.int32(0x007FFFFF)) | jnp.int32(0x3F800000)
    m = jax.lax.bitcast_convert_type(mb, jnp.float32)
    big = m > jnp.float32(_SQRT2)
    e = jnp.where(big, e + 1, e).astype(jnp.float32)
    m = jnp.where(big, m * jnp.float32(0.5), m)
    r = (m - jnp.float32(1.0)) / (m + jnp.float32(1.0))
    r2 = r * r
    p = jnp.float32(2.0 / 9.0)
    p = p * r2 + jnp.float32(2.0 / 7.0)
    p = p * r2 + jnp.float32(2.0 / 5.0)
    p = p * r2 + jnp.float32(2.0 / 3.0)
    p = p * r2 + jnp.float32(2.0)
    p = p * r
    return e * jnp.float32(_LN2_HI) + (e * jnp.float32(_LN2_LO) + p)


def _sc_body(lg_hbm, out_hbm, lg_v, out_v):
    wid = lax.axis_index("s") * 2 + lax.axis_index("c")
    n = ROWS_PER_W * W_SC
    base_off = wid * n
    pltpu.sync_copy(lg_hbm.at[pl.ds(base_off, n)], lg_v)

    lane = jax.lax.broadcasted_iota(jnp.int32, (16,), 0).astype(jnp.uint32)
    sign = jnp.int32(-(2**31))

    for r in range(ROWS_PER_W):
        b = wid * ROWS_PER_W + r
        row_base = (b * (K_SEL * D) + D_TC).astype(jnp.uint32)

        def chunk_step(c, _, row_base=row_base, r=r):
            j0 = c * (16 * CH)
            bases = [row_base + (j0 + 16 * q).astype(jnp.uint32) + lane
                     for q in range(CH)]

            def kstep(k, ms):
                kd = k.astype(jnp.uint32) * jnp.uint32(D)
                out = []
                for q in range(CH):
                    bits = _threefry_xor(bases[q] + kd)
                    s = jax.lax.bitcast_convert_type(bits, jnp.int32) ^ sign
                    out.append(jnp.maximum(ms[q], s))
                return tuple(out)

            ms = jax.lax.fori_loop(
                0, K_SEL, kstep,
                tuple(jnp.full((16,), sign, jnp.int32) for _ in range(CH)))

            for q in range(CH):
                m = jax.lax.bitcast_convert_type(ms[q] ^ sign, jnp.uint32)
                fb = (m >> 9) | jnp.uint32(0x3F800000)
                f = jax.lax.bitcast_convert_type(fb, jnp.float32) - 1.0
                u = jnp.maximum(f, jnp.float32(2.0**-23))
                t = -_poly_log(u)
                gum = -_poly_log(t)
                lgv = lg_v[pl.ds(r * W_SC + j0 + 16 * q, 16)]
                z = (gum + lgv) * jnp.float32(2.0)
                out_v[pl.ds(r * W_SC + j0 + 16 * q, 16)] = (
                    jnp.float32(1.0) / (jnp.float32(1.0) + jnp.exp(-z)))
            return 0

        jax.lax.fori_loop(0, W_SC // (16 * CH), chunk_step, 0)

    pltpu.sync_copy(out_v, out_hbm.at[pl.ds(base_off, n)])


def _sc_part(logits_sc_flat):
    mesh = plsc.VectorSubcoreMesh(core_axis_name="c", subcore_axis_name="s")
    f = functools.partial(
        pl.kernel,
        out_type=jax.ShapeDtypeStruct((B * W_SC,), jnp.float32),
        mesh=mesh,
        scratch_types=[
            pltpu.VMEM((ROWS_PER_W * W_SC,), jnp.float32),
            pltpu.VMEM((ROWS_PER_W * W_SC,), jnp.float32),
        ],
    )(_sc_body)
    return f(logits_sc_flat)


@jax.jit
def kernel(logits):
    lg_sc = logits[:, D_TC:].reshape(-1)
    out_sc = _sc_part(lg_sc).reshape(B, W_SC)
    out_tc = _tc_part(logits[:, :D_TC])
    return jnp.concatenate([out_tc, out_sc], axis=1)


# speedup vs baseline: 1.6903x; 1.0615x over previous
import functools

import jax
import jax.numpy as jnp
import numpy as np
from jax import lax
from jax.experimental import pallas as pl
from jax.experimental.pallas import tpu as pltpu
from jax.experimental.pallas import tpu_sc as plsc

B = 64
K_SEL = 32
D = 8192
TAU0 = 0.5
_TINY = float(np.finfo(np.float32).tiny)

W_SC = 2048
D_TC = D - W_SC
NW = 32
ROWS_PER_W = B // NW
CH = 4

_KS0 = 0
_KS1 = 42
_KS2 = 42 ^ 0x1BD11BDA

_ROT1 = (13, 15, 26, 6)
_ROT2 = (17, 29, 16, 24)


def _round_group(x, y, rots):
    for r in rots:
        x = x + y
        y = (y << r) | (y >> (32 - r))
        y = x ^ y
    return x, y


def _threefry_xor(y):
    u32 = lambda v: jnp.uint32(v)
    x = jnp.zeros_like(y)
    y = y + u32(_KS1)
    x, y = _round_group(x, y, _ROT1)
    x = x + u32(_KS1)
    y = y + u32(_KS2 + 1)
    x, y = _round_group(x, y, _ROT2)
    x = x + u32(_KS2)
    y = y + u32(_KS0 + 2)
    x, y = _round_group(x, y, _ROT1)
    x = x + u32(_KS0)
    y = y + u32(_KS1 + 3)
    x, y = _round_group(x, y, _ROT2)
    x = x + u32(_KS1)
    y = y + u32(_KS2 + 4)
    x, y = _round_group(x, y, _ROT1)
    x = x + u32(_KS2)
    y = y + u32(_KS0 + 5)
    return x ^ y


def _block_kernel(logits_ref, out_ref, *, block_w, rg):
    j = pl.program_id(0)
    c0 = (j * block_w).astype(jnp.uint32)
    row = jax.lax.broadcasted_iota(jnp.uint32, (rg, block_w), 0)
    col = jax.lax.broadcasted_iota(jnp.uint32, (rg, block_w), 1) + c0

    sign = jnp.int32(-(2**31))

    def row_group(g, _):
        b0 = (g * rg).astype(jnp.uint32)
        base = (row + b0) * jnp.uint32(K_SEL * D) + col

        def kstep(k, m):
            bits = _threefry_xor(base + k.astype(jnp.uint32) * jnp.uint32(D))
            return jnp.maximum(m, jax.lax.bitcast_convert_type(bits, jnp.int32) ^ sign)

        m = jax.lax.fori_loop(0, K_SEL, kstep,
                              jnp.full((rg, block_w), sign, jnp.int32),
                              unroll=K_SEL)
        m = jax.lax.bitcast_convert_type(m ^ sign, jnp.uint32)

        fb = (m >> 9) | jnp.uint32(0x3F800000)
        f = jax.lax.bitcast_convert_type(fb, jnp.float32) - 1.0
        u = jnp.maximum(jnp.float32(_TINY),
                        f * jnp.float32(1.0 - _TINY) + jnp.float32(_TINY))
        gum = -jnp.log(-jnp.log(u))
        lg = logits_ref[pl.ds(g * rg, rg), :]
        out_ref[pl.ds(g * rg, rg), :] = jax.nn.sigmoid(
            (gum + lg) * jnp.float32(1.0 / TAU0))
        return 0

    jax.lax.fori_loop(0, B // rg, row_group, 0)


def _tc_part(logits_tc):
    block_w = 1024
    rg = 16
    grid = (D_TC // block_w,)
    return pl.pallas_call(
        functools.partial(_block_kernel, block_w=block_w, rg=rg),
        grid=grid,
        in_specs=[pl.BlockSpec((B, block_w), lambda j: (0, j))],
        out_specs=pl.BlockSpec((B, block_w), lambda j: (0, j)),
        out_shape=jax.ShapeDtypeStruct((B, D_TC), jnp.float32),
    )(logits_tc)


_LN2_HI = 0.693359375
_LN2_LO = -2.12194440e-4
_SQRT2 = 1.4142135623730951


def _poly_log(x):
    xb = jax.lax.bitcast_convert_type(x, jnp.int32)
    e = (xb >> 23) - jnp.int32(127)
    mb = (xb & jnp.int32(0x007FFFFF)) | jnp.int32(0x3F800000)
    m = jax.lax.bitcast_convert_type(mb, jnp.float32)
    big = m > jnp.float32(_SQRT2)
    e = jnp.where(big, e + 1, e).astype(jnp.float32)
    m = jnp.where(big, m * jnp.float32(0.5), m)
    r = (m - jnp.float32(1.0)) / (m + jnp.float32(1.0))
    r2 = r * r
    p = jnp.float32(2.0 / 9.0)
    p = p * r2 + jnp.float32(2.0 / 7.0)
    p = p * r2 + jnp.float32(2.0 / 5.0)
    p = p * r2 + jnp.float32(2.0 / 3.0)
    p = p * r2 + jnp.float32(2.0)
    p = p * r
    return e * jnp.float32(_LN2_HI) + (e * jnp.float32(_LN2_LO) + p)


def _sc_body(lg_hbm, out_hbm, lg_v, out_v):
    wid = lax.axis_index("s") * 2 + lax.axis_index("c")
    n = ROWS_PER_W * W_SC
    base_off = wid * n
    pltpu.sync_copy(lg_hbm.at[pl.ds(base_off, n)], lg_v)

    lane = jax.lax.broadcasted_iota(jnp.int32, (16,), 0).astype(jnp.uint32)
    sign = jnp.int32(-(2**31))

    for r in range(ROWS_PER_W):
        b = wid * ROWS_PER_W + r
        row_base = (b * (K_SEL * D) + D_TC).astype(jnp.uint32)

        def chunk_step(c, _, row_base=row_base, r=r):
            j0 = c * (16 * CH)
            bases = [row_base + (j0 + 16 * q).astype(jnp.uint32) + lane
                     for q in range(CH)]

            def kstep(k, ms):
                kd = k.astype(jnp.uint32) * jnp.uint32(D)
                out = []
                for q in range(CH):
                    bits = _threefry_xor(bases[q] + kd)
                    s = jax.lax.bitcast_convert_type(bits, jnp.int32) ^ sign
                    out.append(jnp.maximum(ms[q], s))
                return tuple(out)

            ms = jax.lax.fori_loop(
                0, K_SEL, kstep,
                tuple(jnp.full((16,), sign, jnp.int32) for _ in range(CH)))

            for q in range(CH):
                m = jax.lax.bitcast_convert_type(ms[q] ^ sign, jnp.uint32)
                fb = (m >> 9) | jnp.uint32(0x3F800000)
                f = jax.lax.bitcast_convert_type(fb, jnp.float32) - 1.0
                u = jnp.maximum(f, jnp.float32(2.0**-23))
                t = -_poly_log(u)
                gum = -_poly_log(t)
                lgv = lg_v[pl.ds(r * W_SC + j0 + 16 * q, 16)]
                z = (gum + lgv) * jnp.float32(2.0)
                out_v[pl.ds(r * W_SC + j0 + 16 * q, 16)] = (
                    jnp.float32(1.0) / (jnp.float32(1.0) + jnp.exp(-z)))
            return 0

        jax.lax.fori_loop(0, W_SC // (16 * CH), chunk_step, 0)

    pltpu.sync_copy(out_v, out_hbm.at[pl.ds(base_off, n)])


def _sc_part(logits_sc_flat):
    mesh = plsc.VectorSubcoreMesh(core_axis_name="c", subcore_axis_name="s")
    f = functools.partial(
        pl.kernel,
        out_type=jax.ShapeDtypeStruct((B * W_SC,), jnp.float32),
        mesh=mesh,
        scratch_types=[
            pltpu.VMEM((ROWS_PER_W * W_SC,), jnp.float32),
            pltpu.VMEM((ROWS_PER_W * W_SC,), jnp.float32),
        ],
    )(_sc_body)
    return f(logits_sc_flat)


@jax.jit
def kernel(logits):
    lg_sc = logits[:, D_TC:].reshape(-1)
    out_sc = _sc_part(lg_sc).reshape(B, W_SC)
    out_tc = _tc_part(logits[:, :D_TC])
    return jnp.concatenate([out_tc, out_sc], axis=1)


# speedup vs baseline: 1.6977x; 1.0044x over previous
import functools

import jax
import jax.numpy as jnp
import numpy as np
from jax import lax
from jax.experimental import pallas as pl
from jax.experimental.pallas import tpu as pltpu
from jax.experimental.pallas import tpu_sc as plsc

B = 64
K_SEL = 32
D = 8192
TAU0 = 0.5
_TINY = float(np.finfo(np.float32).tiny)

W_SC = 2048
D_TC = D - W_SC
NW = 32
ROWS_PER_W = B // NW
CH = 8

_KS0 = 0
_KS1 = 42
_KS2 = 42 ^ 0x1BD11BDA

_ROT1 = (13, 15, 26, 6)
_ROT2 = (17, 29, 16, 24)


def _round_group(x, y, rots):
    for r in rots:
        x = x + y
        y = (y << r) | (y >> (32 - r))
        y = x ^ y
    return x, y


def _threefry_xor(y):
    u32 = lambda v: jnp.uint32(v)
    x = jnp.zeros_like(y)
    y = y + u32(_KS1)
    x, y = _round_group(x, y, _ROT1)
    x = x + u32(_KS1)
    y = y + u32(_KS2 + 1)
    x, y = _round_group(x, y, _ROT2)
    x = x + u32(_KS2)
    y = y + u32(_KS0 + 2)
    x, y = _round_group(x, y, _ROT1)
    x = x + u32(_KS0)
    y = y + u32(_KS1 + 3)
    x, y = _round_group(x, y, _ROT2)
    x = x + u32(_KS1)
    y = y + u32(_KS2 + 4)
    x, y = _round_group(x, y, _ROT1)
    x = x + u32(_KS2)
    y = y + u32(_KS0 + 5)
    return x ^ y


def _block_kernel(logits_ref, out_ref, *, block_w, rg):
    j = pl.program_id(0)
    c0 = (j * block_w).astype(jnp.uint32)
    row = jax.lax.broadcasted_iota(jnp.uint32, (rg, block_w), 0)
    col = jax.lax.broadcasted_iota(jnp.uint32, (rg, block_w), 1) + c0

    sign = jnp.int32(-(2**31))

    def row_group(g, _):
        b0 = (g * rg).astype(jnp.uint32)
        base = (row + b0) * jnp.uint32(K_SEL * D) + col

        def kstep(k, m):
            bits = _threefry_xor(base + k.astype(jnp.uint32) * jnp.uint32(D))
            return jnp.maximum(m, jax.lax.bitcast_convert_type(bits, jnp.int32) ^ sign)

        m = jax.lax.fori_loop(0, K_SEL, kstep,
                              jnp.full((rg, block_w), sign, jnp.int32),
                              unroll=K_SEL)
        m = jax.lax.bitcast_convert_type(m ^ sign, jnp.uint32)

        fb = (m >> 9) | jnp.uint32(0x3F800000)
        f = jax.lax.bitcast_convert_type(fb, jnp.float32) - 1.0
        u = jnp.maximum(jnp.float32(_TINY),
                        f * jnp.float32(1.0 - _TINY) + jnp.float32(_TINY))
        gum = -jnp.log(-jnp.log(u))
        lg = logits_ref[pl.ds(g * rg, rg), :]
        out_ref[pl.ds(g * rg, rg), :] = jax.nn.sigmoid(
            (gum + lg) * jnp.float32(1.0 / TAU0))
        return 0

    jax.lax.fori_loop(0, B // rg, row_group, 0)


def _tc_part(logits_tc):
    block_w = 1024
    rg = 16
    grid = (D_TC // block_w,)
    return pl.pallas_call(
        functools.partial(_block_kernel, block_w=block_w, rg=rg),
        grid=grid,
        in_specs=[pl.BlockSpec((B, block_w), lambda j: (0, j))],
        out_specs=pl.BlockSpec((B, block_w), lambda j: (0, j)),
        out_shape=jax.ShapeDtypeStruct((B, D_TC), jnp.float32),
    )(logits_tc)


_LN2_HI = 0.693359375
_LN2_LO = -2.12194440e-4
_SQRT2 = 1.4142135623730951


def _poly_log(x):
    xb = jax.lax.bitcast_convert_type(x, jnp.int32)
    e = (xb >> 23) - jnp.int32(127)
    mb = (xb & jnp.int32(0x007FFFFF)) | jnp.int32(0x3F800000)
    m = jax.lax.bitcast_convert_type(mb, jnp.float32)
    big = m > jnp.float32(_SQRT2)
    e = jnp.where(big, e + 1, e).astype(jnp.float32)
    m = jnp.where(big, m * jnp.float32(0.5), m)
    r = (m - jnp.float32(1.0)) / (m + jnp.float32(1.0))
    r2 = r * r
    p = jnp.float32(2.0 / 9.0)
    p = p * r2 + jnp.float32(2.0 / 7.0)
    p = p * r2 + jnp.float32(2.0 / 5.0)
    p = p * r2 + jnp.float32(2.0 / 3.0)
    p = p * r2 + jnp.float32(2.0)
    p = p * r
    return e * jnp.float32(_LN2_HI) + (e * jnp.float32(_LN2_LO) + p)


def _sc_body(lg_hbm, out_hbm, lg_v, out_v):
    wid = lax.axis_index("s") * 2 + lax.axis_index("c")
    n = ROWS_PER_W * W_SC
    base_off = wid * n
    pltpu.sync_copy(lg_hbm.at[pl.ds(base_off, n)], lg_v)

    lane = jax.lax.broadcasted_iota(jnp.int32, (16,), 0).astype(jnp.uint32)
    sign = jnp.int32(-(2**31))

    for r in range(ROWS_PER_W):
        b = wid * ROWS_PER_W + r
        row_base = (b * (K_SEL * D) + D_TC).astype(jnp.uint32)

        def chunk_step(c, _, row_base=row_base, r=r):
            j0 = c * (16 * CH)
            bases = [row_base + (j0 + 16 * q).astype(jnp.uint32) + lane
                     for q in range(CH)]

            def kstep(k, ms):
                kd = k.astype(jnp.uint32) * jnp.uint32(D)
                out = []
                for q in range(CH):
                    bits = _threefry_xor(bases[q] + kd)
                    s = jax.lax.bitcast_convert_type(bits, jnp.int32) ^ sign
                    out.append(jnp.maximum(ms[q], s))
                return tuple(out)

            ms = jax.lax.fori_loop(
                0, K_SEL, kstep,
                tuple(jnp.full((16,), sign, jnp.int32) for _ in range(CH)))

            for q in range(CH):
                m = jax.lax.bitcast_convert_type(ms[q] ^ sign, jnp.uint32)
                fb = (m >> 9) | jnp.uint32(0x3F800000)
                f = jax.lax.bitcast_convert_type(fb, jnp.float32) - 1.0
                u = jnp.maximum(f, jnp.float32(2.0**-23))
                t = -_poly_log(u)
                gum = -_poly_log(t)
                lgv = lg_v[pl.ds(r * W_SC + j0 + 16 * q, 16)]
                z = (gum + lgv) * jnp.float32(2.0)
                out_v[pl.ds(r * W_SC + j0 + 16 * q, 16)] = (
                    jnp.float32(1.0) / (jnp.float32(1.0) + jnp.exp(-z)))
            return 0

        jax.lax.fori_loop(0, W_SC // (16 * CH), chunk_step, 0)

    pltpu.sync_copy(out_v, out_hbm.at[pl.ds(base_off, n)])


def _sc_part(logits_sc_flat):
    mesh = plsc.VectorSubcoreMesh(core_axis_name="c", subcore_axis_name="s")
    f = functools.partial(
        pl.kernel,
        out_type=jax.ShapeDtypeStruct((B * W_SC,), jnp.float32),
        mesh=mesh,
        scratch_types=[
            pltpu.VMEM((ROWS_PER_W * W_SC,), jnp.float32),
            pltpu.VMEM((ROWS_PER_W * W_SC,), jnp.float32),
        ],
    )(_sc_body)
    return f(logits_sc_flat)


@jax.jit
def kernel(logits):
    lg_sc = logits[:, D_TC:].reshape(-1)
    out_sc = _sc_part(lg_sc).reshape(B, W_SC)
    out_tc = _tc_part(logits[:, :D_TC])
    return jnp.concatenate([out_tc, out_sc], axis=1)


# speedup vs baseline: 1.7900x; 1.0544x over previous
import functools

import jax
import jax.numpy as jnp
import numpy as np
from jax import lax
from jax.experimental import pallas as pl
from jax.experimental.pallas import tpu as pltpu
from jax.experimental.pallas import tpu_sc as plsc

B = 64
K_SEL = 32
D = 8192
TAU0 = 0.5
_TINY = float(np.finfo(np.float32).tiny)

W_SC = 1920
D_TC = D - W_SC
NW = 32
ROWS_PER_W = B // NW
CH = 8

_KS0 = 0
_KS1 = 42
_KS2 = 42 ^ 0x1BD11BDA

_ROT1 = (13, 15, 26, 6)
_ROT2 = (17, 29, 16, 24)


def _round_group(x, y, rots):
    for r in rots:
        x = x + y
        y = (y << r) | (y >> (32 - r))
        y = x ^ y
    return x, y


def _threefry_xor(y):
    u32 = lambda v: jnp.uint32(v)
    x = jnp.zeros_like(y)
    y = y + u32(_KS1)
    x, y = _round_group(x, y, _ROT1)
    x = x + u32(_KS1)
    y = y + u32(_KS2 + 1)
    x, y = _round_group(x, y, _ROT2)
    x = x + u32(_KS2)
    y = y + u32(_KS0 + 2)
    x, y = _round_group(x, y, _ROT1)
    x = x + u32(_KS0)
    y = y + u32(_KS1 + 3)
    x, y = _round_group(x, y, _ROT2)
    x = x + u32(_KS1)
    y = y + u32(_KS2 + 4)
    x, y = _round_group(x, y, _ROT1)
    x = x + u32(_KS2)
    y = y + u32(_KS0 + 5)
    return x ^ y


def _block_kernel(logits_ref, out_ref, *, block_w, rg):
    j = pl.program_id(0)
    c0 = (j * block_w).astype(jnp.uint32)
    row = jax.lax.broadcasted_iota(jnp.uint32, (rg, block_w), 0)
    col = jax.lax.broadcasted_iota(jnp.uint32, (rg, block_w), 1) + c0

    sign = jnp.int32(-(2**31))

    def row_group(g, _):
        b0 = (g * rg).astype(jnp.uint32)
        base = (row + b0) * jnp.uint32(K_SEL * D) + col

        def kstep(k, m):
            bits = _threefry_xor(base + k.astype(jnp.uint32) * jnp.uint32(D))
            return jnp.maximum(m, jax.lax.bitcast_convert_type(bits, jnp.int32) ^ sign)

        m = jax.lax.fori_loop(0, K_SEL, kstep,
                              jnp.full((rg, block_w), sign, jnp.int32),
                              unroll=K_SEL)
        m = jax.lax.bitcast_convert_type(m ^ sign, jnp.uint32)

        fb = (m >> 9) | jnp.uint32(0x3F800000)
        f = jax.lax.bitcast_convert_type(fb, jnp.float32) - 1.0
        u = jnp.maximum(jnp.float32(_TINY),
                        f * jnp.float32(1.0 - _TINY) + jnp.float32(_TINY))
        gum = -jnp.log(-jnp.log(u))
        lg = logits_ref[pl.ds(g * rg, rg), :]
        out_ref[pl.ds(g * rg, rg), :] = jax.nn.sigmoid(
            (gum + lg) * jnp.float32(1.0 / TAU0))
        return 0

    jax.lax.fori_loop(0, B // rg, row_group, 0)


def _tc_part(logits):
    block_w = D_TC // 7
    rg = 16
    grid = (7,)
    return pl.pallas_call(
        functools.partial(_block_kernel, block_w=block_w, rg=rg),
        grid=grid,
        in_specs=[pl.BlockSpec((B, block_w), lambda j: (0, j))],
        out_specs=pl.BlockSpec((B, block_w), lambda j: (0, j)),
        out_shape=jax.ShapeDtypeStruct((B, D_TC), jnp.float32),
    )(logits)


_LN2_HI = 0.693359375
_LN2_LO = -2.12194440e-4
_SQRT2 = 1.4142135623730951


def _poly_log(x):
    xb = jax.lax.bitcast_convert_type(x, jnp.int32)
    e = (xb >> 23) - jnp.int32(127)
    mb = (xb & jnp.int32(0x007FFFFF)) | jnp.int32(0x3F800000)
    m = jax.lax.bitcast_convert_type(mb, jnp.float32)
    big = m > jnp.float32(_SQRT2)
    e = jnp.where(big, e + 1, e).astype(jnp.float32)
    m = jnp.where(big, m * jnp.float32(0.5), m)
    r = (m - jnp.float32(1.0)) / (m + jnp.float32(1.0))
    r2 = r * r
    p = jnp.float32(2.0 / 9.0)
    p = p * r2 + jnp.float32(2.0 / 7.0)
    p = p * r2 + jnp.float32(2.0 / 5.0)
    p = p * r2 + jnp.float32(2.0 / 3.0)
    p = p * r2 + jnp.float32(2.0)
    p = p * r
    return e * jnp.float32(_LN2_HI) + (e * jnp.float32(_LN2_LO) + p)


def _sc_body(lg_hbm, out_hbm, lg_v, out_v):
    wid = lax.axis_index("s") * 2 + lax.axis_index("c")
    n = ROWS_PER_W * W_SC
    base_off = wid * n
    for r in range(ROWS_PER_W):
        b = wid * ROWS_PER_W + r
        pltpu.sync_copy(lg_hbm.at[pl.ds(b * D + D_TC, W_SC)],
                        lg_v.at[pl.ds(r * W_SC, W_SC)])

    lane = jax.lax.broadcasted_iota(jnp.int32, (16,), 0).astype(jnp.uint32)
    sign = jnp.int32(-(2**31))

    for r in range(ROWS_PER_W):
        b = wid * ROWS_PER_W + r
        row_base = (b * (K_SEL * D) + D_TC).astype(jnp.uint32)

        def chunk_step(c, _, row_base=row_base, r=r):
            j0 = c * (16 * CH)
            bases = [row_base + (j0 + 16 * q).astype(jnp.uint32) + lane
                     for q in range(CH)]

            def kstep(k, ms):
                kd = k.astype(jnp.uint32) * jnp.uint32(D)
                out = []
                for q in range(CH):
                    bits = _threefry_xor(bases[q] + kd)
                    s = jax.lax.bitcast_convert_type(bits, jnp.int32) ^ sign
                    out.append(jnp.maximum(ms[q], s))
                return tuple(out)

            ms = jax.lax.fori_loop(
                0, K_SEL, kstep,
                tuple(jnp.full((16,), sign, jnp.int32) for _ in range(CH)))

            for q in range(CH):
                m = jax.lax.bitcast_convert_type(ms[q] ^ sign, jnp.uint32)
                fb = (m >> 9) | jnp.uint32(0x3F800000)
                f = jax.lax.bitcast_convert_type(fb, jnp.float32) - 1.0
                u = jnp.maximum(f, jnp.float32(2.0**-23))
                t = -_poly_log(u)
                gum = -_poly_log(t)
                lgv = lg_v[pl.ds(r * W_SC + j0 + 16 * q, 16)]
                z = (gum + lgv) * jnp.float32(2.0)
                out_v[pl.ds(r * W_SC + j0 + 16 * q, 16)] = (
                    jnp.float32(1.0) / (jnp.float32(1.0) + jnp.exp(-z)))
            return 0

        jax.lax.fori_loop(0, W_SC // (16 * CH), chunk_step, 0)

    pltpu.sync_copy(out_v, out_hbm.at[pl.ds(base_off, n)])


def _sc_part(logits_sc_flat):
    mesh = plsc.VectorSubcoreMesh(core_axis_name="c", subcore_axis_name="s")
    f = functools.partial(
        pl.kernel,
        out_type=jax.ShapeDtypeStruct((B * W_SC,), jnp.float32),
        mesh=mesh,
        scratch_types=[
            pltpu.VMEM((ROWS_PER_W * W_SC,), jnp.float32),
            pltpu.VMEM((ROWS_PER_W * W_SC,), jnp.float32),
        ],
    )(_sc_body)
    return f(logits_sc_flat)


@jax.jit
def kernel(logits):
    out_sc = _sc_part(logits.reshape(-1)).reshape(B, W_SC)
    out_tc = _tc_part(logits)
    return jnp.concatenate([out_tc, out_sc], axis=1)


# speedup vs baseline: 1.8210x; 1.0173x over previous
import functools

import jax
import jax.numpy as jnp
import numpy as np
from jax import lax
from jax.experimental import pallas as pl
from jax.experimental.pallas import tpu as pltpu
from jax.experimental.pallas import tpu_sc as plsc

B = 64
K_SEL = 32
D = 8192
TAU0 = 0.5
_TINY = float(np.finfo(np.float32).tiny)

W_SC = 1920
D_TC = D - W_SC
NW = 32
ROWS_PER_W = B // NW
CH = 8

_KS0 = 0
_KS1 = 42
_KS2 = 42 ^ 0x1BD11BDA

_ROT1 = (13, 15, 26, 6)
_ROT2 = (17, 29, 16, 24)


def _round_group(x, y, rots):
    for r in rots:
        x = x + y
        y = (y << r) | (y >> (32 - r))
        y = x ^ y
    return x, y


def _threefry_xor(y):
    u32 = lambda v: jnp.uint32(v)
    x = jnp.zeros_like(y)
    y = y + u32(_KS1)
    x, y = _round_group(x, y, _ROT1)
    x = x + u32(_KS1)
    y = y + u32(_KS2 + 1)
    x, y = _round_group(x, y, _ROT2)
    x = x + u32(_KS2)
    y = y + u32(_KS0 + 2)
    x, y = _round_group(x, y, _ROT1)
    x = x + u32(_KS0)
    y = y + u32(_KS1 + 3)
    x, y = _round_group(x, y, _ROT2)
    x = x + u32(_KS1)
    y = y + u32(_KS2 + 4)
    x, y = _round_group(x, y, _ROT1)
    x = x + u32(_KS2)
    y = y + u32(_KS0 + 5)
    return x ^ y


def _block_kernel(logits_ref, out_ref, *, block_w, rg):
    j = pl.program_id(0)
    c0 = (j * block_w).astype(jnp.uint32)
    row = jax.lax.broadcasted_iota(jnp.uint32, (rg, block_w), 0)
    col = jax.lax.broadcasted_iota(jnp.uint32, (rg, block_w), 1) + c0

    sign = jnp.int32(-(2**31))

    def row_group(g, _):
        b0 = (g * rg).astype(jnp.uint32)
        base = (row + b0) * jnp.uint32(K_SEL * D) + col

        def kstep(k, m):
            bits = _threefry_xor(base + k.astype(jnp.uint32) * jnp.uint32(D))
            return jnp.maximum(m, jax.lax.bitcast_convert_type(bits, jnp.int32) ^ sign)

        m = jax.lax.fori_loop(0, K_SEL, kstep,
                              jnp.full((rg, block_w), sign, jnp.int32),
                              unroll=K_SEL)
        m = jax.lax.bitcast_convert_type(m ^ sign, jnp.uint32)

        fb = (m >> 9) | jnp.uint32(0x3F800000)
        f = jax.lax.bitcast_convert_type(fb, jnp.float32) - 1.0
        u = jnp.maximum(jnp.float32(_TINY),
                        f * jnp.float32(1.0 - _TINY) + jnp.float32(_TINY))
        gum = -jnp.log(-jnp.log(u))
        lg = logits_ref[pl.ds(g * rg, rg), :]
        out_ref[pl.ds(g * rg, rg), :] = jax.nn.sigmoid(
            (gum + lg) * jnp.float32(1.0 / TAU0))
        return 0

    jax.lax.fori_loop(0, B // rg, row_group, 0)


def _tc_part(logits):
    block_w = D_TC // 7
    rg = 16
    grid = (7,)
    return pl.pallas_call(
        functools.partial(_block_kernel, block_w=block_w, rg=rg),
        grid=grid,
        in_specs=[pl.BlockSpec((B, block_w), lambda j: (0, j))],
        out_specs=pl.BlockSpec((B, block_w), lambda j: (0, j)),
        out_shape=jax.ShapeDtypeStruct((B, D_TC), jnp.float32),
    )(logits)


_LN2_HI = 0.693359375
_LN2_LO = -2.12194440e-4
_SQRT2 = 1.4142135623730951


def _poly_log(x):
    xb = jax.lax.bitcast_convert_type(x, jnp.int32)
    e = (xb >> 23) - jnp.int32(127)
    mb = (xb & jnp.int32(0x007FFFFF)) | jnp.int32(0x3F800000)
    m = jax.lax.bitcast_convert_type(mb, jnp.float32)
    big = m > jnp.float32(_SQRT2)
    e = jnp.where(big, e + 1, e).astype(jnp.float32)
    m = jnp.where(big, m * jnp.float32(0.5), m)
    r = (m - jnp.float32(1.0)) / (m + jnp.float32(1.0))
    r2 = r * r
    p = jnp.float32(2.0 / 9.0)
    p = p * r2 + jnp.float32(2.0 / 7.0)
    p = p * r2 + jnp.float32(2.0 / 5.0)
    p = p * r2 + jnp.float32(2.0 / 3.0)
    p = p * r2 + jnp.float32(2.0)
    p = p * r
    return e * jnp.float32(_LN2_HI) + (e * jnp.float32(_LN2_LO) + p)


def _sc_body(lg_hbm, out_hbm, lg_v, out_v):
    wid = lax.axis_index("s") * 2 + lax.axis_index("c")
    for r in range(ROWS_PER_W):
        b = wid * ROWS_PER_W + r
        pltpu.sync_copy(lg_hbm.at[b, pl.ds(D_TC, W_SC)],
                        lg_v.at[pl.ds(r * W_SC, W_SC)])

    lane = jax.lax.broadcasted_iota(jnp.int32, (16,), 0).astype(jnp.uint32)
    sign = jnp.int32(-(2**31))

    for r in range(ROWS_PER_W):
        b = wid * ROWS_PER_W + r
        row_base = (b * (K_SEL * D) + D_TC).astype(jnp.uint32)

        def chunk_step(c, _, row_base=row_base, r=r):
            j0 = c * (16 * CH)
            bases = [row_base + (j0 + 16 * q).astype(jnp.uint32) + lane
                     for q in range(CH)]

            def kstep(k, ms):
                kd = k.astype(jnp.uint32) * jnp.uint32(D)
                out = []
                for q in range(CH):
                    bits = _threefry_xor(bases[q] + kd)
                    s = jax.lax.bitcast_convert_type(bits, jnp.int32) ^ sign
                    out.append(jnp.maximum(ms[q], s))
                return tuple(out)

            ms = jax.lax.fori_loop(
                0, K_SEL, kstep,
                tuple(jnp.full((16,), sign, jnp.int32) for _ in range(CH)))

            for q in range(CH):
                m = jax.lax.bitcast_convert_type(ms[q] ^ sign, jnp.uint32)
                fb = (m >> 9) | jnp.uint32(0x3F800000)
                f = jax.lax.bitcast_convert_type(fb, jnp.float32) - 1.0
                u = jnp.maximum(f, jnp.float32(2.0**-23))
                t = -_poly_log(u)
                gum = -_poly_log(t)
                lgv = lg_v[pl.ds(r * W_SC + j0 + 16 * q, 16)]
                z = (gum + lgv) * jnp.float32(2.0)
                out_v[pl.ds(r * W_SC + j0 + 16 * q, 16)] = (
                    jnp.float32(1.0) / (jnp.float32(1.0) + jnp.exp(-z)))
            return 0

        jax.lax.fori_loop(0, W_SC // (16 * CH), chunk_step, 0)

    for r in range(ROWS_PER_W):
        b = wid * ROWS_PER_W + r
        pltpu.sync_copy(out_v.at[pl.ds(r * W_SC, W_SC)], out_hbm.at[b])


def _sc_part(logits):
    mesh = plsc.VectorSubcoreMesh(core_axis_name="c", subcore_axis_name="s")
    f = functools.partial(
        pl.kernel,
        out_type=jax.ShapeDtypeStruct((B, W_SC), jnp.float32),
        mesh=mesh,
        scratch_types=[
            pltpu.VMEM((ROWS_PER_W * W_SC,), jnp.float32),
            pltpu.VMEM((ROWS_PER_W * W_SC,), jnp.float32),
        ],
    )(_sc_body)
    return f(logits)


@jax.jit
def kernel(logits):
    out_sc = _sc_part(logits)
    out_tc = _tc_part(logits)
    return jnp.concatenate([out_tc, out_sc], axis=1)
